# Initial kernel scaffold; baseline (speedup 1.0000x reference)
import jax
import jax.numpy as jnp
from jax.experimental import pallas as pl


def kernel(token_embeddings, pos_table):
    raise NotImplementedError("write your pallas kernel here")


# speedup vs baseline: 1.2755x; 1.2755x over previous
import jax
import jax.numpy as jnp
from jax.experimental import pallas as pl


def _add_body(tok_ref, pos_ref, out_ref):
    out_ref[...] = tok_ref[...] + pos_ref[...]


def kernel(token_embeddings, pos_table):
    if token_embeddings.ndim == 2:
        token_embeddings = token_embeddings[None, :, :]
    B, S, D = token_embeddings.shape

    S_BLK = 512
    grid = (B, S // S_BLK)

    out = pl.pallas_call(
        _add_body,
        grid=grid,
        in_specs=[
            pl.BlockSpec((1, S_BLK, D), lambda b, s: (b, s, 0)),
            pl.BlockSpec((S_BLK, D), lambda b, s: (s, 0)),
        ],
        out_specs=pl.BlockSpec((1, S_BLK, D), lambda b, s: (b, s, 0)),
        out_shape=jax.ShapeDtypeStruct((B, S, D), token_embeddings.dtype),
    )(token_embeddings, pos_table[:S])
    return out
